# Initial kernel scaffold; baseline (speedup 1.0000x reference)
import jax
import jax.numpy as jnp
from jax.experimental import pallas as pl


def kernel(input_h, relation_embed, edges, W_neigh0, W_loop0, W_neigh1, W_loop1):
    raise NotImplementedError("write your pallas kernel here")


# speedup vs baseline: 2.7608x; 2.7608x over previous
import functools

import jax
import jax.numpy as jnp
from jax import lax
from jax.experimental import pallas as pl
from jax.experimental.pallas import tpu as pltpu
from jax.experimental.pallas import tpu_sc as plsc

_N = 10000
_DIM = 128
_NPAD = 10112
_NC = 2
_NS = 16
_NW = _NC * _NS
_CHUNK = 128
_G = 8
_RPT = _NPAD // _NS
_DEGW = 16
_SLOPE = (1.0 / 8.0 + 1.0 / 3.0) / 2.0
_ROWBLK = 1264

_BLKS = [(o, min(_CHUNK, _RPT - o)) for o in range(0, _RPT, _CHUNK)]


def _zero_slice(zeros_v, sh, r0):
    for off, blk in _BLKS:
        pltpu.sync_copy(zeros_v.at[pl.ds(0, blk)], sh.at[pl.ds(r0 + off, blk)])


def _dump_slice(sh, bounce_v, out_hbm, cid, r0):
    for off, blk in _BLKS:
        pltpu.sync_copy(sh.at[pl.ds(r0 + off, blk)], bounce_v.at[pl.ds(0, blk)])
        pltpu.sync_copy(bounce_v.at[pl.ds(0, blk)], out_hbm.at[cid, pl.ds(r0 + off, blk)])


def _make_sc_first(nch):
    mesh = plsc.VectorSubcoreMesh(core_axis_name="c", subcore_axis_name="s")
    ngrp = nch // _G

    @functools.partial(
        pl.kernel,
        mesh=mesh,
        out_type=[
            jax.ShapeDtypeStruct((_NC, _NPAD, _DIM), jnp.float32),
            jax.ShapeDtypeStruct((_NC, _NPAD, _DIM), jnp.float32),
            jax.ShapeDtypeStruct((_NC, _NPAD, _DIM), jnp.float32),
        ],
        scratch_types=[
            pltpu.VMEM((_G, _CHUNK), jnp.int32),
            pltpu.VMEM((_G, _CHUNK), jnp.int32),
            pltpu.VMEM((_CHUNK, _DIM), jnp.float32),
            pltpu.VMEM_SHARED((_NPAD, _DIM), jnp.float32),
            pltpu.SemaphoreType.DMA,
        ],
    )
    def sc_first(h_hbm, r_hbm, src_hbm, rel_hbm, dst_hbm, zrow_hbm,
                 ones_hbm, aggh_out, aggr_out, deg_out,
                 idx_v, dst_v, rows_v, acc_sh, sem):
        cid = lax.axis_index("c")
        sid = lax.axis_index("s")
        wid = sid * _NC + cid
        r0 = sid * _RPT

        pltpu.sync_copy(zrow_hbm, rows_v)
        _zero_slice(rows_v, acc_sh, r0)
        plsc.subcore_barrier()

        def grp_r(g, carry):
            pltpu.sync_copy(rel_hbm.at[wid, pl.ds(g * _G, _G)], idx_v)
            pltpu.sync_copy(dst_hbm.at[wid, pl.ds(g * _G, _G)], dst_v)
            for j in range(_G):
                pltpu.async_copy(r_hbm.at[idx_v.at[j]], rows_v, sem).wait()
                pltpu.sync_copy(rows_v, acc_sh.at[dst_v.at[j]], add=True)
            return carry

        lax.fori_loop(0, ngrp, grp_r, 0)
        plsc.subcore_barrier()
        _dump_slice(acc_sh, rows_v, aggr_out, cid, r0)
        pltpu.sync_copy(zrow_hbm, rows_v)
        _zero_slice(rows_v, acc_sh, r0)
        plsc.subcore_barrier()

        def grp_h(g, carry):
            pltpu.sync_copy(src_hbm.at[wid, pl.ds(g * _G, _G)], idx_v)
            pltpu.sync_copy(dst_hbm.at[wid, pl.ds(g * _G, _G)], dst_v)
            for j in range(_G):
                pltpu.async_copy(h_hbm.at[idx_v.at[j]], rows_v, sem).wait()
                pltpu.sync_copy(rows_v, acc_sh.at[dst_v.at[j]], add=True)
            return carry

        lax.fori_loop(0, ngrp, grp_h, 0)
        plsc.subcore_barrier()
        _dump_slice(acc_sh, rows_v, aggh_out, cid, r0)
        pltpu.sync_copy(zrow_hbm, rows_v)
        _zero_slice(rows_v, acc_sh, r0)
        plsc.subcore_barrier()

        pltpu.sync_copy(ones_hbm, rows_v)

        def grp_d(g, carry):
            pltpu.sync_copy(dst_hbm.at[wid, pl.ds(g * _G, _G)], dst_v)
            for j in range(_G):
                pltpu.sync_copy(rows_v, acc_sh.at[dst_v.at[j]], add=True)
            return carry

        lax.fori_loop(0, ngrp, grp_d, 0)
        plsc.subcore_barrier()
        _dump_slice(acc_sh, rows_v, deg_out, cid, r0)

    return sc_first


def _make_sc_second(nch):
    mesh = plsc.VectorSubcoreMesh(core_axis_name="c", subcore_axis_name="s")
    ngrp = nch // _G

    @functools.partial(
        pl.kernel,
        mesh=mesh,
        out_type=[
            jax.ShapeDtypeStruct((_NC, _NPAD, _DIM), jnp.float32),
        ],
        scratch_types=[
            pltpu.VMEM((_G, _CHUNK), jnp.int32),
            pltpu.VMEM((_G, _CHUNK), jnp.int32),
            pltpu.VMEM((_CHUNK, _DIM), jnp.float32),
            pltpu.VMEM_SHARED((_NPAD, _DIM), jnp.float32),
            pltpu.SemaphoreType.DMA,
        ],
    )
    def sc_second(h_hbm, src_hbm, dst_hbm, zrow_hbm, aggh_out,
                  src_v, dst_v, rows_v, acc_sh, sem):
        cid = lax.axis_index("c")
        sid = lax.axis_index("s")
        wid = sid * _NC + cid
        r0 = sid * _RPT

        pltpu.sync_copy(zrow_hbm, rows_v)
        _zero_slice(rows_v, acc_sh, r0)
        plsc.subcore_barrier()

        def grp_h(g, carry):
            pltpu.sync_copy(src_hbm.at[wid, pl.ds(g * _G, _G)], src_v)
            pltpu.sync_copy(dst_hbm.at[wid, pl.ds(g * _G, _G)], dst_v)
            for j in range(_G):
                pltpu.async_copy(h_hbm.at[src_v.at[j]], rows_v, sem).wait()
                pltpu.sync_copy(rows_v, acc_sh.at[dst_v.at[j]], add=True)
            return carry

        lax.fori_loop(0, ngrp, grp_h, 0)
        plsc.subcore_barrier()
        _dump_slice(acc_sh, rows_v, aggh_out, cid, r0)

    return sc_second


def _tc_layer(acch, accr, degp, h, w_neigh, w_loop):
    def body(acch_ref, accr_ref, deg_ref, h_ref, wn_ref, wl_ref, out_ref):
        s = acch_ref[0] + acch_ref[1] + accr_ref[0] + accr_ref[1]
        d = deg_ref[0, :, 0:1] + deg_ref[1, :, 0:1]
        agg = s / jnp.maximum(d, 1.0)
        out = (jnp.dot(agg, wn_ref[...], preferred_element_type=jnp.float32)
               + jnp.dot(h_ref[...], wl_ref[...], preferred_element_type=jnp.float32))
        out_ref[...] = jnp.where(out >= 0, out, _SLOPE * out)

    return pl.pallas_call(
        body,
        grid=(_NPAD // _ROWBLK,),
        in_specs=[
            pl.BlockSpec((_NC, _ROWBLK, _DIM), lambda i: (0, i, 0)),
            pl.BlockSpec((_NC, _ROWBLK, _DIM), lambda i: (0, i, 0)),
            pl.BlockSpec((_NC, _ROWBLK, _DIM), lambda i: (0, i, 0)),
            pl.BlockSpec((_ROWBLK, _DIM), lambda i: (i, 0)),
            pl.BlockSpec((_DIM, _DIM), lambda i: (0, 0)),
            pl.BlockSpec((_DIM, _DIM), lambda i: (0, 0)),
        ],
        out_specs=pl.BlockSpec((_ROWBLK, _DIM), lambda i: (i, 0)),
        out_shape=jax.ShapeDtypeStruct((_NPAD, _DIM), jnp.float32),
    )(acch, accr, degp, h, w_neigh, w_loop)


def kernel(input_h, relation_embed, edges, W_neigh0, W_loop0, W_neigh1, W_loop1):
    e = edges.astype(jnp.int32)
    n_edges = e.shape[0]
    nch = -(-n_edges // (_NW * _CHUNK * _G)) * _G
    epad = _NW * _CHUNK * nch
    pad = epad - n_edges
    src = jnp.concatenate([e[:, 0], jnp.zeros((pad,), jnp.int32)])
    rel = jnp.concatenate([e[:, 1], jnp.zeros((pad,), jnp.int32)])
    dst = jnp.concatenate([e[:, 2], jnp.full((pad,), _NPAD - 1, jnp.int32)])
    src = src.reshape(_NW, nch, _CHUNK)
    rel = rel.reshape(_NW, nch, _CHUNK)
    dst = dst.reshape(_NW, nch, _CHUNK)

    h0 = jnp.pad(input_h, ((0, _NPAD - _N), (0, 0)))
    zrow = jnp.zeros((_CHUNK, _DIM), jnp.float32)
    ones = jnp.ones((_CHUNK, _DIM), jnp.float32)

    aggh0, aggr, degp = _make_sc_first(nch)(
        h0, relation_embed, src, rel, dst, zrow, ones)
    h1 = _tc_layer(aggh0, aggr, degp, h0, W_neigh0, W_loop0)
    aggh1, = _make_sc_second(nch)(h1, src, dst, zrow)
    h2 = _tc_layer(aggh1, aggr, degp, h1, W_neigh1, W_loop1)
    return h2[:_N]


# speedup vs baseline: 3.0131x; 1.0914x over previous
import functools

import jax
import jax.numpy as jnp
from jax import lax
from jax.experimental import pallas as pl
from jax.experimental.pallas import tpu as pltpu
from jax.experimental.pallas import tpu_sc as plsc

_N = 10000
_DIM = 128
_NPAD = 10112
_NC = 2
_NS = 16
_NW = _NC * _NS
_CHUNK = 128
_G = 8
_RPT = _NPAD // _NS
_DEGW = 16
_SLOPE = (1.0 / 8.0 + 1.0 / 3.0) / 2.0
_ROWBLK = 1264

_BLKS = [(o, min(_CHUNK, _RPT - o)) for o in range(0, _RPT, _CHUNK)]


def _zero_slice(zeros_v, sh, r0):
    for off, blk in _BLKS:
        pltpu.sync_copy(zeros_v.at[pl.ds(0, blk)], sh.at[pl.ds(r0 + off, blk)])


def _dump_slice(sh, bounce_v, out_hbm, cid, r0):
    for off, blk in _BLKS:
        pltpu.sync_copy(sh.at[pl.ds(r0 + off, blk)], bounce_v.at[pl.ds(0, blk)])
        pltpu.sync_copy(bounce_v.at[pl.ds(0, blk)], out_hbm.at[cid, pl.ds(r0 + off, blk)])


def _make_sc_first(nch):
    mesh = plsc.VectorSubcoreMesh(core_axis_name="c", subcore_axis_name="s")
    ngrp = nch // _G

    @functools.partial(
        pl.kernel,
        mesh=mesh,
        out_type=[
            jax.ShapeDtypeStruct((_NC, _NPAD, _DIM), jnp.float32),
            jax.ShapeDtypeStruct((_NC, _NPAD, _DIM), jnp.float32),
            jax.ShapeDtypeStruct((_NC, _NPAD, _DIM), jnp.float32),
        ],
        scratch_types=[
            pltpu.VMEM((_G, _CHUNK), jnp.int32),
            pltpu.VMEM((_G, _CHUNK), jnp.int32),
            pltpu.VMEM((_CHUNK, _DIM), jnp.float32),
            pltpu.VMEM((_CHUNK, _DIM), jnp.float32),
            pltpu.VMEM_SHARED((_NPAD, _DIM), jnp.float32),
            pltpu.SemaphoreType.DMA,
            pltpu.SemaphoreType.DMA,
        ],
    )
    def sc_first(h_hbm, r_hbm, src_hbm, rel_hbm, dst_hbm, zrow_hbm,
                 ones_hbm, aggh_out, aggr_out, deg_out,
                 idx_v, dst_v, rows_v, rows2_v, acc_sh, semA, semB):
        cid = lax.axis_index("c")
        sid = lax.axis_index("s")
        wid = sid * _NC + cid
        r0 = sid * _RPT

        pltpu.sync_copy(zrow_hbm, rows_v)
        _zero_slice(rows_v, acc_sh, r0)
        plsc.subcore_barrier()

        bufs = (rows_v, rows2_v)
        sems = (semA, semB)

        def make_grp(idx_hbm, tab_hbm):
            def grp(g, carry):
                pltpu.sync_copy(idx_hbm.at[wid, pl.ds(g * _G, _G)], idx_v)
                pltpu.sync_copy(dst_hbm.at[wid, pl.ds(g * _G, _G)], dst_v)
                cp = [None, None]
                cp[0] = pltpu.async_copy(tab_hbm.at[idx_v.at[0]], bufs[0], sems[0])
                for j in range(_G):
                    k = j % 2
                    if j + 1 < _G:
                        cp[1 - k] = pltpu.async_copy(
                            tab_hbm.at[idx_v.at[j + 1]], bufs[1 - k], sems[1 - k])
                    cp[k].wait()
                    pltpu.sync_copy(bufs[k], acc_sh.at[dst_v.at[j]], add=True)
                return carry
            return grp

        lax.fori_loop(0, ngrp, make_grp(rel_hbm, r_hbm), 0)
        plsc.subcore_barrier()
        _dump_slice(acc_sh, rows_v, aggr_out, cid, r0)
        pltpu.sync_copy(zrow_hbm, rows_v)
        _zero_slice(rows_v, acc_sh, r0)
        plsc.subcore_barrier()

        lax.fori_loop(0, ngrp, make_grp(src_hbm, h_hbm), 0)
        plsc.subcore_barrier()
        _dump_slice(acc_sh, rows_v, aggh_out, cid, r0)
        pltpu.sync_copy(zrow_hbm, rows_v)
        _zero_slice(rows_v, acc_sh, r0)
        plsc.subcore_barrier()

        pltpu.sync_copy(ones_hbm, rows_v)

        def grp_d(g, carry):
            pltpu.sync_copy(dst_hbm.at[wid, pl.ds(g * _G, _G)], dst_v)
            for j in range(_G):
                pltpu.sync_copy(rows_v, acc_sh.at[dst_v.at[j]], add=True)
            return carry

        lax.fori_loop(0, ngrp, grp_d, 0)
        plsc.subcore_barrier()
        _dump_slice(acc_sh, rows_v, deg_out, cid, r0)

    return sc_first


def _make_sc_second(nch):
    mesh = plsc.VectorSubcoreMesh(core_axis_name="c", subcore_axis_name="s")
    ngrp = nch // _G

    @functools.partial(
        pl.kernel,
        mesh=mesh,
        out_type=[
            jax.ShapeDtypeStruct((_NC, _NPAD, _DIM), jnp.float32),
        ],
        scratch_types=[
            pltpu.VMEM((_G, _CHUNK), jnp.int32),
            pltpu.VMEM((_G, _CHUNK), jnp.int32),
            pltpu.VMEM((_CHUNK, _DIM), jnp.float32),
            pltpu.VMEM((_CHUNK, _DIM), jnp.float32),
            pltpu.VMEM_SHARED((_NPAD, _DIM), jnp.float32),
            pltpu.SemaphoreType.DMA,
            pltpu.SemaphoreType.DMA,
        ],
    )
    def sc_second(h_hbm, src_hbm, dst_hbm, zrow_hbm, aggh_out,
                  src_v, dst_v, rows_v, rows2_v, acc_sh, semA, semB):
        cid = lax.axis_index("c")
        sid = lax.axis_index("s")
        wid = sid * _NC + cid
        r0 = sid * _RPT

        pltpu.sync_copy(zrow_hbm, rows_v)
        _zero_slice(rows_v, acc_sh, r0)
        plsc.subcore_barrier()

        bufs = (rows_v, rows2_v)
        sems = (semA, semB)

        def grp_h(g, carry):
            pltpu.sync_copy(src_hbm.at[wid, pl.ds(g * _G, _G)], src_v)
            pltpu.sync_copy(dst_hbm.at[wid, pl.ds(g * _G, _G)], dst_v)
            cp = [None, None]
            cp[0] = pltpu.async_copy(h_hbm.at[src_v.at[0]], bufs[0], sems[0])
            for j in range(_G):
                k = j % 2
                if j + 1 < _G:
                    cp[1 - k] = pltpu.async_copy(
                        h_hbm.at[src_v.at[j + 1]], bufs[1 - k], sems[1 - k])
                cp[k].wait()
                pltpu.sync_copy(bufs[k], acc_sh.at[dst_v.at[j]], add=True)
            return carry

        lax.fori_loop(0, ngrp, grp_h, 0)
        plsc.subcore_barrier()
        _dump_slice(acc_sh, rows_v, aggh_out, cid, r0)

    return sc_second


def _tc_layer(acch, accr, degp, h, w_neigh, w_loop):
    def body(acch_ref, accr_ref, deg_ref, h_ref, wn_ref, wl_ref, out_ref):
        s = acch_ref[0] + acch_ref[1] + accr_ref[0] + accr_ref[1]
        d = deg_ref[0, :, 0:1] + deg_ref[1, :, 0:1]
        agg = s / jnp.maximum(d, 1.0)
        out = (jnp.dot(agg, wn_ref[...], preferred_element_type=jnp.float32)
               + jnp.dot(h_ref[...], wl_ref[...], preferred_element_type=jnp.float32))
        out_ref[...] = jnp.where(out >= 0, out, _SLOPE * out)

    return pl.pallas_call(
        body,
        grid=(_NPAD // _ROWBLK,),
        in_specs=[
            pl.BlockSpec((_NC, _ROWBLK, _DIM), lambda i: (0, i, 0)),
            pl.BlockSpec((_NC, _ROWBLK, _DIM), lambda i: (0, i, 0)),
            pl.BlockSpec((_NC, _ROWBLK, _DIM), lambda i: (0, i, 0)),
            pl.BlockSpec((_ROWBLK, _DIM), lambda i: (i, 0)),
            pl.BlockSpec((_DIM, _DIM), lambda i: (0, 0)),
            pl.BlockSpec((_DIM, _DIM), lambda i: (0, 0)),
        ],
        out_specs=pl.BlockSpec((_ROWBLK, _DIM), lambda i: (i, 0)),
        out_shape=jax.ShapeDtypeStruct((_NPAD, _DIM), jnp.float32),
    )(acch, accr, degp, h, w_neigh, w_loop)


def kernel(input_h, relation_embed, edges, W_neigh0, W_loop0, W_neigh1, W_loop1):
    e = edges.astype(jnp.int32)
    n_edges = e.shape[0]
    nch = -(-n_edges // (_NW * _CHUNK * _G)) * _G
    epad = _NW * _CHUNK * nch
    pad = epad - n_edges
    src = jnp.concatenate([e[:, 0], jnp.zeros((pad,), jnp.int32)])
    rel = jnp.concatenate([e[:, 1], jnp.zeros((pad,), jnp.int32)])
    dst = jnp.concatenate([e[:, 2], jnp.full((pad,), _NPAD - 1, jnp.int32)])
    src = src.reshape(_NW, nch, _CHUNK)
    rel = rel.reshape(_NW, nch, _CHUNK)
    dst = dst.reshape(_NW, nch, _CHUNK)

    h0 = jnp.pad(input_h, ((0, _NPAD - _N), (0, 0)))
    zrow = jnp.zeros((_CHUNK, _DIM), jnp.float32)
    ones = jnp.ones((_CHUNK, _DIM), jnp.float32)

    aggh0, aggr, degp = _make_sc_first(nch)(
        h0, relation_embed, src, rel, dst, zrow, ones)
    h1 = _tc_layer(aggh0, aggr, degp, h0, W_neigh0, W_loop0)
    aggh1, = _make_sc_second(nch)(h1, src, dst, zrow)
    h2 = _tc_layer(aggh1, aggr, degp, h1, W_neigh1, W_loop1)
    return h2[:_N]


# speedup vs baseline: 8.8070x; 2.9229x over previous
import functools

import jax
import jax.numpy as jnp
from jax import lax
from jax.experimental import pallas as pl
from jax.experimental.pallas import tpu as pltpu
from jax.experimental.pallas import tpu_sc as plsc

_N = 10000
_DIM = 128
_NPAD = 10112
_NC = 2
_NS = 16
_NW = _NC * _NS
_CHUNK = 128
_G = 8
_RPT = _NPAD // _NS
_DEGW = 16
_SLOPE = (1.0 / 8.0 + 1.0 / 3.0) / 2.0
_ROWBLK = 1264

_BLKS = [(o, min(_CHUNK, _RPT - o)) for o in range(0, _RPT, _CHUNK)]


def _zero_slice(zeros_v, sh, r0):
    for off, blk in _BLKS:
        pltpu.sync_copy(zeros_v.at[pl.ds(0, blk)], sh.at[pl.ds(r0 + off, blk)])


def _dump_slice(sh, bounce_v, out_hbm, cid, r0):
    for off, blk in _BLKS:
        pltpu.sync_copy(sh.at[pl.ds(r0 + off, blk)], bounce_v.at[pl.ds(0, blk)])
        pltpu.sync_copy(bounce_v.at[pl.ds(0, blk)], out_hbm.at[cid, pl.ds(r0 + off, blk)])


def _make_sc_first(nch):
    mesh = plsc.VectorSubcoreMesh(core_axis_name="c", subcore_axis_name="s")
    ngrp = nch // _G

    @functools.partial(
        pl.kernel,
        mesh=mesh,
        out_type=[
            jax.ShapeDtypeStruct((_NC, _NPAD, _DIM), jnp.float32),
            jax.ShapeDtypeStruct((_NC, _NPAD, _DIM), jnp.float32),
            jax.ShapeDtypeStruct((_NC, _NPAD, _DIM), jnp.float32),
        ],
        scratch_types=[
            pltpu.VMEM((_G, _CHUNK), jnp.int32),
            pltpu.VMEM((_G, _CHUNK), jnp.int32),
            pltpu.VMEM((_CHUNK, _DIM), jnp.float32),
            pltpu.VMEM((_CHUNK, _DIM), jnp.float32),
            pltpu.VMEM_SHARED((_NPAD, _DIM), jnp.float32),
            pltpu.SemaphoreType.DMA,
            pltpu.SemaphoreType.DMA,
        ],
    )
    def sc_first(h_hbm, r_hbm, src_hbm, rel_hbm, dst_hbm, zrow_hbm,
                 ones_hbm, aggh_out, aggr_out, deg_out,
                 idx_v, dst_v, rows_v, rows2_v, acc_sh, semA, semB):
        cid = lax.axis_index("c")
        sid = lax.axis_index("s")
        wid = sid * _NC + cid
        r0 = sid * _RPT

        pltpu.sync_copy(zrow_hbm, rows_v)
        _zero_slice(rows_v, acc_sh, r0)
        plsc.subcore_barrier()

        bufs = (rows_v, rows2_v)
        sems = (semA, semB)

        def make_grp(idx_hbm, tab_hbm):
            def grp(g, carry):
                pltpu.sync_copy(idx_hbm.at[wid, pl.ds(g * _G, _G)], idx_v)
                pltpu.sync_copy(dst_hbm.at[wid, pl.ds(g * _G, _G)], dst_v)
                cp = [None, None]
                cp[0] = pltpu.async_copy(tab_hbm.at[idx_v.at[0]], bufs[0], sems[0])
                for j in range(_G):
                    k = j % 2
                    if j + 1 < _G:
                        cp[1 - k] = pltpu.async_copy(
                            tab_hbm.at[idx_v.at[j + 1]], bufs[1 - k], sems[1 - k])
                    cp[k].wait()
                    pltpu.sync_copy(bufs[k], acc_sh.at[dst_v.at[j]], add=True)
                return carry
            return grp

        lax.fori_loop(0, ngrp, make_grp(rel_hbm, r_hbm), 0)
        plsc.subcore_barrier()
        _dump_slice(acc_sh, rows_v, aggr_out, cid, r0)
        pltpu.sync_copy(zrow_hbm, rows_v)
        _zero_slice(rows_v, acc_sh, r0)
        plsc.subcore_barrier()

        lax.fori_loop(0, ngrp, make_grp(src_hbm, h_hbm), 0)
        plsc.subcore_barrier()
        _dump_slice(acc_sh, rows_v, aggh_out, cid, r0)
        pltpu.sync_copy(zrow_hbm, rows_v)
        _zero_slice(rows_v, acc_sh, r0)
        plsc.subcore_barrier()

        pltpu.sync_copy(ones_hbm, rows_v)

        def grp_d(g, carry):
            pltpu.sync_copy(dst_hbm.at[wid, pl.ds(g * _G, _G)], dst_v)
            for j in range(_G):
                pltpu.sync_copy(rows_v, acc_sh.at[dst_v.at[j]], add=True)
            return carry

        lax.fori_loop(0, ngrp, grp_d, 0)
        plsc.subcore_barrier()
        _dump_slice(acc_sh, rows_v, deg_out, cid, r0)

    return sc_first


def _make_sc_second(nch):
    mesh = plsc.VectorSubcoreMesh(core_axis_name="c", subcore_axis_name="s")
    ngrp = nch // _G

    @functools.partial(
        pl.kernel,
        mesh=mesh,
        out_type=[
            jax.ShapeDtypeStruct((_NC, _NPAD, _DIM), jnp.float32),
        ],
        scratch_types=[
            pltpu.VMEM((_G, _CHUNK), jnp.int32),
            pltpu.VMEM((_G, _CHUNK), jnp.int32),
            pltpu.VMEM((_CHUNK, _DIM), jnp.float32),
            pltpu.VMEM((_CHUNK, _DIM), jnp.float32),
            pltpu.VMEM_SHARED((_NPAD, _DIM), jnp.float32),
            pltpu.SemaphoreType.DMA,
            pltpu.SemaphoreType.DMA,
        ],
    )
    def sc_second(h_hbm, src_hbm, dst_hbm, zrow_hbm, aggh_out,
                  src_v, dst_v, rows_v, rows2_v, acc_sh, semA, semB):
        cid = lax.axis_index("c")
        sid = lax.axis_index("s")
        wid = sid * _NC + cid
        r0 = sid * _RPT

        pltpu.sync_copy(zrow_hbm, rows_v)
        _zero_slice(rows_v, acc_sh, r0)
        plsc.subcore_barrier()

        bufs = (rows_v, rows2_v)
        sems = (semA, semB)

        def grp_h(g, carry):
            pltpu.sync_copy(src_hbm.at[wid, pl.ds(g * _G, _G)], src_v)
            pltpu.sync_copy(dst_hbm.at[wid, pl.ds(g * _G, _G)], dst_v)
            cp = [None, None]
            cp[0] = pltpu.async_copy(h_hbm.at[src_v.at[0]], bufs[0], sems[0])
            for j in range(_G):
                k = j % 2
                if j + 1 < _G:
                    cp[1 - k] = pltpu.async_copy(
                        h_hbm.at[src_v.at[j + 1]], bufs[1 - k], sems[1 - k])
                cp[k].wait()
                pltpu.sync_copy(bufs[k], acc_sh.at[dst_v.at[j]], add=True)
            return carry

        lax.fori_loop(0, ngrp, grp_h, 0)
        plsc.subcore_barrier()
        _dump_slice(acc_sh, rows_v, aggh_out, cid, r0)

    return sc_second


def _tc_layer(acch, accr, degp, h, w_neigh, w_loop):
    def body(acch_ref, accr_ref, deg_ref, h_ref, wn_ref, wl_ref, out_ref):
        s = acch_ref[0] + acch_ref[1] + accr_ref[0] + accr_ref[1]
        d = deg_ref[0, :, 0:1] + deg_ref[1, :, 0:1]
        agg = s / jnp.maximum(d, 1.0)
        out = (jnp.dot(agg, wn_ref[...], preferred_element_type=jnp.float32)
               + jnp.dot(h_ref[...], wl_ref[...], preferred_element_type=jnp.float32))
        out_ref[...] = jnp.where(out >= 0, out, _SLOPE * out)

    return pl.pallas_call(
        body,
        grid=(_NPAD // _ROWBLK,),
        in_specs=[
            pl.BlockSpec((_NC, _ROWBLK, _DIM), lambda i: (0, i, 0)),
            pl.BlockSpec((_NC, _ROWBLK, _DIM), lambda i: (0, i, 0)),
            pl.BlockSpec((_NC, _ROWBLK, _DIM), lambda i: (0, i, 0)),
            pl.BlockSpec((_ROWBLK, _DIM), lambda i: (i, 0)),
            pl.BlockSpec((_DIM, _DIM), lambda i: (0, 0)),
            pl.BlockSpec((_DIM, _DIM), lambda i: (0, 0)),
        ],
        out_specs=pl.BlockSpec((_ROWBLK, _DIM), lambda i: (i, 0)),
        out_shape=jax.ShapeDtypeStruct((_NPAD, _DIM), jnp.float32),
    )(acch, accr, degp, h, w_neigh, w_loop)


def kernel(input_h, relation_embed, edges, W_neigh0, W_loop0, W_neigh1, W_loop1):
    e = edges.astype(jnp.int32)
    n_edges = e.shape[0]
    nch = -(-n_edges // (_NW * _CHUNK * _G)) * _G
    epad = _NW * _CHUNK * nch
    pad = epad - n_edges
    fill = jnp.arange(pad, dtype=jnp.int32)
    src = jnp.concatenate([e[:, 0], fill % _N])
    rel = jnp.concatenate([e[:, 1], fill % _N])
    dst = jnp.concatenate([e[:, 2], _N + fill % (_NPAD - _N)])
    src = src.reshape(nch, _NW, _CHUNK).transpose(1, 0, 2)
    rel = rel.reshape(nch, _NW, _CHUNK).transpose(1, 0, 2)
    dst = dst.reshape(nch, _NW, _CHUNK).transpose(1, 0, 2)

    h0 = jnp.pad(input_h, ((0, _NPAD - _N), (0, 0)))
    zrow = jnp.zeros((_CHUNK, _DIM), jnp.float32)
    ones = jnp.ones((_CHUNK, _DIM), jnp.float32)

    aggh0, aggr, degp = _make_sc_first(nch)(
        h0, relation_embed, src, rel, dst, zrow, ones)
    h1 = _tc_layer(aggh0, aggr, degp, h0, W_neigh0, W_loop0)
    aggh1, = _make_sc_second(nch)(h1, src, dst, zrow)
    h2 = _tc_layer(aggh1, aggr, degp, h1, W_neigh1, W_loop1)
    return h2[:_N]


# speedup vs baseline: 8.9057x; 1.0112x over previous
import functools

import jax
import jax.numpy as jnp
from jax import lax
from jax.experimental import pallas as pl
from jax.experimental.pallas import tpu as pltpu
from jax.experimental.pallas import tpu_sc as plsc

_N = 10000
_DIM = 128
_NPAD = 10112
_NC = 2
_NS = 16
_NW = _NC * _NS
_CHUNK = 128
_G = 8
_RPT = _NPAD // _NS
_DEGW = 16
_SLOPE = (1.0 / 8.0 + 1.0 / 3.0) / 2.0
_ROWBLK = 1264

_BLKS = [(o, min(_CHUNK, _RPT - o)) for o in range(0, _RPT, _CHUNK)]


def _zero_slice(zeros_v, sh, r0):
    for off, blk in _BLKS:
        pltpu.sync_copy(zeros_v.at[pl.ds(0, blk)], sh.at[pl.ds(r0 + off, blk)])


def _dump_slice(sh, bounce_v, out_hbm, cid, r0):
    for off, blk in _BLKS:
        pltpu.sync_copy(sh.at[pl.ds(r0 + off, blk)], bounce_v.at[pl.ds(0, blk)])
        pltpu.sync_copy(bounce_v.at[pl.ds(0, blk)], out_hbm.at[cid, pl.ds(r0 + off, blk)])


def _load_slice(in_hbm, bounce_v, sh, cid, r0):
    for off, blk in _BLKS:
        pltpu.sync_copy(in_hbm.at[cid, pl.ds(r0 + off, blk)], bounce_v.at[pl.ds(0, blk)])
        pltpu.sync_copy(bounce_v.at[pl.ds(0, blk)], sh.at[pl.ds(r0 + off, blk)])


def _make_sc_first(nch):
    mesh = plsc.VectorSubcoreMesh(core_axis_name="c", subcore_axis_name="s")
    ngrp = nch // _G

    @functools.partial(
        pl.kernel,
        mesh=mesh,
        out_type=[
            jax.ShapeDtypeStruct((_NC, _NPAD, _DIM), jnp.float32),
            jax.ShapeDtypeStruct((_NC, _NPAD, _DIM), jnp.float32),
            jax.ShapeDtypeStruct((_NC, _NPAD, _DIM), jnp.float32),
        ],
        scratch_types=[
            pltpu.VMEM((_G, _CHUNK), jnp.int32),
            pltpu.VMEM((_G, _CHUNK), jnp.int32),
            pltpu.VMEM((_CHUNK, _DIM), jnp.float32),
            pltpu.VMEM((_CHUNK, _DIM), jnp.float32),
            pltpu.VMEM_SHARED((_NPAD, _DIM), jnp.float32),
            pltpu.SemaphoreType.DMA,
            pltpu.SemaphoreType.DMA,
        ],
    )
    def sc_first(h_hbm, r_hbm, src_hbm, rel_hbm, dst_hbm, zrow_hbm,
                 ones_hbm, aggh_out, aggr_out, deg_out,
                 idx_v, dst_v, rows_v, rows2_v, acc_sh, semA, semB):
        cid = lax.axis_index("c")
        sid = lax.axis_index("s")
        wid = sid * _NC + cid
        r0 = sid * _RPT

        pltpu.sync_copy(zrow_hbm, rows_v)
        _zero_slice(rows_v, acc_sh, r0)
        plsc.subcore_barrier()

        bufs = (rows_v, rows2_v)
        sems = (semA, semB)

        def make_grp(idx_hbm, tab_hbm):
            def grp(g, carry):
                pltpu.sync_copy(idx_hbm.at[wid, pl.ds(g * _G, _G)], idx_v)
                pltpu.sync_copy(dst_hbm.at[wid, pl.ds(g * _G, _G)], dst_v)
                cp = [None, None]
                cp[0] = pltpu.async_copy(tab_hbm.at[idx_v.at[0]], bufs[0], sems[0])
                for j in range(_G):
                    k = j % 2
                    if j + 1 < _G:
                        cp[1 - k] = pltpu.async_copy(
                            tab_hbm.at[idx_v.at[j + 1]], bufs[1 - k], sems[1 - k])
                    cp[k].wait()
                    pltpu.sync_copy(bufs[k], acc_sh.at[dst_v.at[j]], add=True)
                return carry
            return grp

        lax.fori_loop(0, ngrp, make_grp(rel_hbm, r_hbm), 0)
        plsc.subcore_barrier()
        _dump_slice(acc_sh, rows_v, aggr_out, cid, r0)
        plsc.subcore_barrier()

        lax.fori_loop(0, ngrp, make_grp(src_hbm, h_hbm), 0)
        plsc.subcore_barrier()
        _dump_slice(acc_sh, rows_v, aggh_out, cid, r0)
        pltpu.sync_copy(zrow_hbm, rows_v)
        _zero_slice(rows_v, acc_sh, r0)
        plsc.subcore_barrier()

        pltpu.sync_copy(ones_hbm, rows_v)

        def grp_d(g, carry):
            pltpu.sync_copy(dst_hbm.at[wid, pl.ds(g * _G, _G)], dst_v)
            for j in range(_G):
                pltpu.sync_copy(rows_v, acc_sh.at[dst_v.at[j]], add=True)
            return carry

        lax.fori_loop(0, ngrp, grp_d, 0)
        plsc.subcore_barrier()
        _dump_slice(acc_sh, rows_v, deg_out, cid, r0)

    return sc_first


def _make_sc_second(nch):
    mesh = plsc.VectorSubcoreMesh(core_axis_name="c", subcore_axis_name="s")
    ngrp = nch // _G

    @functools.partial(
        pl.kernel,
        mesh=mesh,
        out_type=[
            jax.ShapeDtypeStruct((_NC, _NPAD, _DIM), jnp.float32),
        ],
        scratch_types=[
            pltpu.VMEM((_G, _CHUNK), jnp.int32),
            pltpu.VMEM((_G, _CHUNK), jnp.int32),
            pltpu.VMEM((_CHUNK, _DIM), jnp.float32),
            pltpu.VMEM((_CHUNK, _DIM), jnp.float32),
            pltpu.VMEM_SHARED((_NPAD, _DIM), jnp.float32),
            pltpu.SemaphoreType.DMA,
            pltpu.SemaphoreType.DMA,
        ],
    )
    def sc_second(h_hbm, src_hbm, dst_hbm, aggr_hbm, aggh_out,
                  src_v, dst_v, rows_v, rows2_v, acc_sh, semA, semB):
        cid = lax.axis_index("c")
        sid = lax.axis_index("s")
        wid = sid * _NC + cid
        r0 = sid * _RPT

        _load_slice(aggr_hbm, rows_v, acc_sh, cid, r0)
        plsc.subcore_barrier()

        bufs = (rows_v, rows2_v)
        sems = (semA, semB)

        def grp_h(g, carry):
            pltpu.sync_copy(src_hbm.at[wid, pl.ds(g * _G, _G)], src_v)
            pltpu.sync_copy(dst_hbm.at[wid, pl.ds(g * _G, _G)], dst_v)
            cp = [None, None]
            cp[0] = pltpu.async_copy(h_hbm.at[src_v.at[0]], bufs[0], sems[0])
            for j in range(_G):
                k = j % 2
                if j + 1 < _G:
                    cp[1 - k] = pltpu.async_copy(
                        h_hbm.at[src_v.at[j + 1]], bufs[1 - k], sems[1 - k])
                cp[k].wait()
                pltpu.sync_copy(bufs[k], acc_sh.at[dst_v.at[j]], add=True)
            return carry

        lax.fori_loop(0, ngrp, grp_h, 0)
        plsc.subcore_barrier()
        _dump_slice(acc_sh, rows_v, aggh_out, cid, r0)

    return sc_second


def _tc_layer(acch, degp, h, w_neigh, w_loop):
    def body(acch_ref, deg_ref, h_ref, wn_ref, wl_ref, out_ref):
        s = acch_ref[0] + acch_ref[1]
        d = deg_ref[0, :, 0:1] + deg_ref[1, :, 0:1]
        agg = s / jnp.maximum(d, 1.0)
        out = (jnp.dot(agg, wn_ref[...], preferred_element_type=jnp.float32)
               + jnp.dot(h_ref[...], wl_ref[...], preferred_element_type=jnp.float32))
        out_ref[...] = jnp.where(out >= 0, out, _SLOPE * out)

    return pl.pallas_call(
        body,
        grid=(_NPAD // _ROWBLK,),
        in_specs=[
            pl.BlockSpec((_NC, _ROWBLK, _DIM), lambda i: (0, i, 0)),
            pl.BlockSpec((_NC, _ROWBLK, _DIM), lambda i: (0, i, 0)),
            pl.BlockSpec((_ROWBLK, _DIM), lambda i: (i, 0)),
            pl.BlockSpec((_DIM, _DIM), lambda i: (0, 0)),
            pl.BlockSpec((_DIM, _DIM), lambda i: (0, 0)),
        ],
        out_specs=pl.BlockSpec((_ROWBLK, _DIM), lambda i: (i, 0)),
        out_shape=jax.ShapeDtypeStruct((_NPAD, _DIM), jnp.float32),
    )(acch, degp, h, w_neigh, w_loop)


def kernel(input_h, relation_embed, edges, W_neigh0, W_loop0, W_neigh1, W_loop1):
    e = edges.astype(jnp.int32)
    n_edges = e.shape[0]
    nch = -(-n_edges // (_NW * _CHUNK * _G)) * _G
    epad = _NW * _CHUNK * nch
    pad = epad - n_edges
    fill = jnp.arange(pad, dtype=jnp.int32)
    src = jnp.concatenate([e[:, 0], fill % _N])
    rel = jnp.concatenate([e[:, 1], fill % _N])
    dst = jnp.concatenate([e[:, 2], _N + fill % (_NPAD - _N)])
    src = src.reshape(nch, _NW, _CHUNK).transpose(1, 0, 2)
    rel = rel.reshape(nch, _NW, _CHUNK).transpose(1, 0, 2)
    dst = dst.reshape(nch, _NW, _CHUNK).transpose(1, 0, 2)

    h0 = jnp.pad(input_h, ((0, _NPAD - _N), (0, 0)))
    zrow = jnp.zeros((_CHUNK, _DIM), jnp.float32)
    ones = jnp.ones((_CHUNK, _DIM), jnp.float32)

    aggm0, aggr, degp = _make_sc_first(nch)(
        h0, relation_embed, src, rel, dst, zrow, ones)
    h1 = _tc_layer(aggm0, degp, h0, W_neigh0, W_loop0)
    aggm1, = _make_sc_second(nch)(h1, src, dst, aggr)
    h2 = _tc_layer(aggm1, degp, h1, W_neigh1, W_loop1)
    return h2[:_N]
